# Initial kernel scaffold; baseline (speedup 1.0000x reference)
import functools

import jax
import jax.numpy as jnp
from jax import lax
from jax.experimental import pallas as pl
from jax.experimental.pallas import tpu as pltpu
from jax.experimental.pallas import tpu_sc as plsc

N_NODES = 10000
N_EDGES = 320000
D_EDGE = 16
EMB = 128

NC = 2
NS = 16
NW = NC * NS
EPW = N_EDGES // NW
CH = 80
NCHUNK = EPW // CH
RPT = N_NODES // NS

_EB = 4000
_RB = 1000


def _edge_embed(edge_attr, Wcat, bcat):
  def body(a_ref, w_ref, b_ref, o1_ref, o2_ref):
    e = jnp.dot(a_ref[...], w_ref[...],
                preferred_element_type=jnp.float32) + b_ref[...]
    o1_ref[...] = e[:, :EMB]
    o2_ref[...] = e[:, EMB:]

  return pl.pallas_call(
      body,
      grid=(N_EDGES // _EB,),
      in_specs=[
          pl.BlockSpec((_EB, D_EDGE), lambda i: (i, 0)),
          pl.BlockSpec((D_EDGE, 2 * EMB), lambda i: (0, 0)),
          pl.BlockSpec((1, 2 * EMB), lambda i: (0, 0)),
      ],
      out_specs=[
          pl.BlockSpec((_EB, EMB), lambda i: (i, 0)),
          pl.BlockSpec((_EB, EMB), lambda i: (i, 0)),
      ],
      out_shape=[jax.ShapeDtypeStruct((N_EDGES, EMB), jnp.float32)] * 2,
  )(edge_attr, Wcat, bcat)


_SC_MESH = plsc.VectorSubcoreMesh(core_axis_name="c", subcore_axis_name="s")


@functools.partial(
    pl.kernel,
    out_type=jax.ShapeDtypeStruct((NC * N_NODES, EMB), jnp.float32),
    mesh=_SC_MESH,
    scratch_types=[
        pltpu.VMEM((NCHUNK, CH), jnp.int32),
        pltpu.VMEM((NCHUNK, CH), jnp.int32),
        pltpu.VMEM((CH, EMB), jnp.float32),
        pltpu.VMEM((CH, EMB), jnp.float32),
        pltpu.VMEM_SHARED((N_NODES, EMB), jnp.float32),
        pltpu.SemaphoreType.DMA,
    ],
)
def _sc_segment(h_hbm, e_hbm, src_hbm, dst_hbm, z_hbm, out_hbm,
                sidx, didx, hrows, erows, aggsh, sem):
  c = lax.axis_index("c")
  s = lax.axis_index("s")
  wid = c * NS + s

  pltpu.sync_copy(z_hbm, aggsh.at[pl.ds(s * RPT, RPT)])
  pltpu.sync_copy(src_hbm.at[wid], sidx)
  pltpu.sync_copy(dst_hbm.at[wid], didx)
  plsc.subcore_barrier()

  @pl.loop(0, NCHUNK)
  def _chunk(t):
    base = wid * EPW + t * CH
    gat = pltpu.async_copy(h_hbm.at[sidx.at[t]], hrows, sem)
    pltpu.sync_copy(e_hbm.at[pl.ds(base, CH)], erows)
    gat.wait()

    @pl.loop(0, CH)
    def _row(r):
      for j in range(EMB // 16):
        sl = pl.ds(j * 16, 16)
        hrows[r, sl] = jnp.maximum(hrows[r, sl] + erows[r, sl], 0.0)

    pltpu.sync_copy(hrows, aggsh.at[didx.at[t]], add=True)

  plsc.subcore_barrier()
  pltpu.sync_copy(aggsh.at[pl.ds(s * RPT, RPT)],
                  out_hbm.at[pl.ds(c * N_NODES + s * RPT, RPT)])


def _mlp(h, parts, alpha, W1f, b1f, W2f, b2f, relu_out):
  def body(al_ref, h_ref, p_ref, w1_ref, b1_ref, w2_ref, b2_ref, o_ref):
    t = h_ref[...] * al_ref[0, 0] + p_ref[0] + p_ref[1]
    t = jnp.dot(t, w1_ref[...], preferred_element_type=jnp.float32) + b1_ref[...]
    t = jnp.maximum(t, 0.0)
    t = jnp.dot(t, w2_ref[...], preferred_element_type=jnp.float32) + b2_ref[...]
    if relu_out:
      t = jnp.maximum(t, 0.0)
    o_ref[...] = t

  return pl.pallas_call(
      body,
      grid=(N_NODES // _RB,),
      in_specs=[
          pl.BlockSpec((1, 1), lambda i: (0, 0)),
          pl.BlockSpec((_RB, EMB), lambda i: (i, 0)),
          pl.BlockSpec((NC, _RB, EMB), lambda i: (0, i, 0)),
          pl.BlockSpec((EMB, 2 * EMB), lambda i: (0, 0)),
          pl.BlockSpec((1, 2 * EMB), lambda i: (0, 0)),
          pl.BlockSpec((2 * EMB, EMB), lambda i: (0, 0)),
          pl.BlockSpec((1, EMB), lambda i: (0, 0)),
      ],
      out_specs=pl.BlockSpec((_RB, EMB), lambda i: (i, 0)),
      out_shape=jax.ShapeDtypeStruct((N_NODES, EMB), jnp.float32),
  )(alpha, h, parts, W1f, b1f, W2f, b2f)


def kernel(x, edge_index, edge_attr, We, be, eps, W1, b1, W2, b2,
           g1, bb1, m1, v1, go, bo, mo, vo):
  s1 = g1 / jnp.sqrt(v1 + 1e-5)
  W1f = W1 * s1[:, None, :]
  b1f = (b1 - m1) * s1 + bb1
  so = go / jnp.sqrt(vo + 1e-5)
  W2f = W2 * so[:, None, :]
  b2f = (b2 - mo) * so + bo

  Wcat = jnp.concatenate([We[0], We[1]], axis=1)
  bcat = jnp.concatenate([be[0], be[1]])[None, :]
  E1, E2 = _edge_embed(edge_attr, Wcat, bcat)

  src = edge_index[0].reshape(NW, NCHUNK, CH)
  dst = edge_index[1].reshape(NW, NCHUNK, CH)
  z = jnp.zeros((RPT, EMB), jnp.float32)

  h = x
  for l in range(2):
    El = E1 if l == 0 else E2
    parts = _sc_segment(h, El, src, dst, z).reshape(NC, N_NODES, EMB)
    alpha = (1.0 + eps[l]).reshape(1, 1)
    h = _mlp(h, parts, alpha, W1f[l], b1f[l], W2f[l], b2f[l],
             relu_out=(l == 0))
  return h


# speedup vs baseline: 2.8559x; 2.8559x over previous
import functools

import jax
import jax.numpy as jnp
from jax import lax
from jax.experimental import pallas as pl
from jax.experimental.pallas import tpu as pltpu
from jax.experimental.pallas import tpu_sc as plsc

N_NODES = 10000
N_EDGES = 320000
D_EDGE = 16
EMB = 128

NC = 2
NS = 16
NW = NC * NS
EPW = N_EDGES // NW
CH = 80
NCHUNK = EPW // CH
NPAD = 10240
RPT = NPAD // NS

_EB = 4000
_RB = 1000


def _edge_embed(edge_attr, Wcat, bcat):
  def body(a_ref, w_ref, b_ref, o1_ref, o2_ref):
    e = jnp.dot(a_ref[...], w_ref[...],
                preferred_element_type=jnp.float32) + b_ref[...]
    o1_ref[...] = e[:, :EMB]
    o2_ref[...] = e[:, EMB:]

  return pl.pallas_call(
      body,
      grid=(N_EDGES // _EB,),
      in_specs=[
          pl.BlockSpec((_EB, D_EDGE), lambda i: (i, 0)),
          pl.BlockSpec((D_EDGE, 2 * EMB), lambda i: (0, 0)),
          pl.BlockSpec((1, 2 * EMB), lambda i: (0, 0)),
      ],
      out_specs=[
          pl.BlockSpec((_EB, EMB), lambda i: (i, 0)),
          pl.BlockSpec((_EB, EMB), lambda i: (i, 0)),
      ],
      out_shape=[jax.ShapeDtypeStruct((N_EDGES, EMB), jnp.float32)] * 2,
  )(edge_attr, Wcat, bcat)


_SC_MESH = plsc.VectorSubcoreMesh(core_axis_name="c", subcore_axis_name="s")


@functools.partial(
    pl.kernel,
    out_type=jax.ShapeDtypeStruct((NC * NPAD, EMB), jnp.float32),
    mesh=_SC_MESH,
    scratch_types=[
        pltpu.VMEM((CH,), jnp.int32),
        pltpu.VMEM((CH,), jnp.int32),
        pltpu.VMEM((CH, EMB), jnp.float32),
        pltpu.VMEM((CH, EMB), jnp.float32),
        pltpu.VMEM_SHARED((NPAD, EMB), jnp.float32),
        pltpu.SemaphoreType.DMA,
    ],
)
def _sc_segment(h_hbm, e_hbm, src_hbm, dst_hbm, z_hbm, out_hbm,
                sidx, didx, hrows, erows, aggsh, sem):
  c = lax.axis_index("c")
  s = lax.axis_index("s")
  wid = c * NS + s

  pltpu.sync_copy(z_hbm, aggsh.at[pl.ds(s * RPT, RPT)])
  plsc.subcore_barrier()

  @pl.loop(0, NCHUNK)
  def _chunk(t):
    base = wid * EPW + t * CH
    pltpu.sync_copy(src_hbm.at[pl.ds(base, CH)], sidx)
    pltpu.sync_copy(dst_hbm.at[pl.ds(base, CH)], didx)
    gat = pltpu.async_copy(h_hbm.at[sidx], hrows, sem)
    pltpu.sync_copy(e_hbm.at[pl.ds(base, CH)], erows)
    gat.wait()

    @pl.loop(0, CH)
    def _row(r):
      for j in range(EMB // 16):
        sl = pl.ds(j * 16, 16)
        hrows[r, sl] = jnp.maximum(hrows[r, sl] + erows[r, sl], 0.0)

    pltpu.sync_copy(hrows, aggsh.at[didx], add=True)

  plsc.subcore_barrier()
  pltpu.sync_copy(aggsh.at[pl.ds(s * RPT, RPT)],
                  out_hbm.at[pl.ds(c * NPAD + s * RPT, RPT)])


def _mlp(h, parts, alpha, W1f, b1f, W2f, b2f, relu_out):
  def body(al_ref, h_ref, p_ref, w1_ref, b1_ref, w2_ref, b2_ref, o_ref):
    t = h_ref[...] * al_ref[0, 0] + p_ref[0] + p_ref[1]
    t = jnp.dot(t, w1_ref[...], preferred_element_type=jnp.float32) + b1_ref[...]
    t = jnp.maximum(t, 0.0)
    t = jnp.dot(t, w2_ref[...], preferred_element_type=jnp.float32) + b2_ref[...]
    if relu_out:
      t = jnp.maximum(t, 0.0)
    o_ref[...] = t

  return pl.pallas_call(
      body,
      grid=(N_NODES // _RB,),
      in_specs=[
          pl.BlockSpec((1, 1), lambda i: (0, 0)),
          pl.BlockSpec((_RB, EMB), lambda i: (i, 0)),
          pl.BlockSpec((NC, _RB, EMB), lambda i: (0, i, 0)),
          pl.BlockSpec((EMB, 2 * EMB), lambda i: (0, 0)),
          pl.BlockSpec((1, 2 * EMB), lambda i: (0, 0)),
          pl.BlockSpec((2 * EMB, EMB), lambda i: (0, 0)),
          pl.BlockSpec((1, EMB), lambda i: (0, 0)),
      ],
      out_specs=pl.BlockSpec((_RB, EMB), lambda i: (i, 0)),
      out_shape=jax.ShapeDtypeStruct((N_NODES, EMB), jnp.float32),
  )(alpha, h, parts, W1f, b1f[None], W2f, b2f[None])


def kernel(x, edge_index, edge_attr, We, be, eps, W1, b1, W2, b2,
           g1, bb1, m1, v1, go, bo, mo, vo):
  s1 = g1 / jnp.sqrt(v1 + 1e-5)
  W1f = W1 * s1[:, None, :]
  b1f = (b1 - m1) * s1 + bb1
  so = go / jnp.sqrt(vo + 1e-5)
  W2f = W2 * so[:, None, :]
  b2f = (b2 - mo) * so + bo

  Wcat = jnp.concatenate([We[0], We[1]], axis=1)
  bcat = jnp.concatenate([be[0], be[1]])[None, :]
  E1, E2 = _edge_embed(edge_attr, Wcat, bcat)

  src = edge_index[0]
  dst = edge_index[1]
  z = jnp.zeros((RPT, EMB), jnp.float32)

  h = x
  for l in range(2):
    El = E1 if l == 0 else E2
    parts = _sc_segment(h, El, src, dst, z).reshape(NC, NPAD, EMB)
    alpha = (1.0 + eps[l]).reshape(1, 1)
    h = _mlp(h, parts, alpha, W1f[l], b1f[l], W2f[l], b2f[l],
             relu_out=(l == 0))
  return h


# speedup vs baseline: 4.4686x; 1.5647x over previous
import functools

import jax
import jax.numpy as jnp
from jax import lax
from jax.experimental import pallas as pl
from jax.experimental.pallas import tpu as pltpu
from jax.experimental.pallas import tpu_sc as plsc

N_NODES = 10000
N_EDGES = 320000
D_EDGE = 16
EMB = 128

NC = 2
NS = 16
NW = NC * NS
EPW = N_EDGES // NW
CH = 80
NCHUNK = EPW // CH
NPAD = 10240
RPT = NPAD // NS

_EB = 4000
_RB = 1000


def _edge_embed(edge_attr, Wcat, bcat):
  def body(a_ref, w_ref, b_ref, o1_ref, o2_ref):
    e = jnp.dot(a_ref[...], w_ref[...],
                preferred_element_type=jnp.float32) + b_ref[...]
    o1_ref[...] = e[:, :EMB]
    o2_ref[...] = e[:, EMB:]

  return pl.pallas_call(
      body,
      grid=(N_EDGES // _EB,),
      in_specs=[
          pl.BlockSpec((_EB, D_EDGE), lambda i: (i, 0)),
          pl.BlockSpec((D_EDGE, 2 * EMB), lambda i: (0, 0)),
          pl.BlockSpec((1, 2 * EMB), lambda i: (0, 0)),
      ],
      out_specs=[
          pl.BlockSpec((_EB, EMB), lambda i: (i, 0)),
          pl.BlockSpec((_EB, EMB), lambda i: (i, 0)),
      ],
      out_shape=[jax.ShapeDtypeStruct((N_EDGES, EMB), jnp.float32)] * 2,
  )(edge_attr, Wcat, bcat)


_SC_MESH = plsc.VectorSubcoreMesh(core_axis_name="c", subcore_axis_name="s")


@functools.partial(
    pl.kernel,
    out_type=jax.ShapeDtypeStruct((NC * NPAD, EMB), jnp.float32),
    mesh=_SC_MESH,
    scratch_types=[
        pltpu.VMEM((2, CH), jnp.int32),
        pltpu.VMEM((2, CH), jnp.int32),
        pltpu.VMEM((2, CH, EMB), jnp.float32),
        pltpu.VMEM((2, CH, EMB), jnp.float32),
        pltpu.VMEM_SHARED((NPAD, EMB), jnp.float32),
        [pltpu.SemaphoreType.DMA] * 2,
        [pltpu.SemaphoreType.DMA] * 2,
        [pltpu.SemaphoreType.DMA] * 2,
    ],
)
def _sc_segment(h_hbm, e_hbm, src_hbm, dst_hbm, z_hbm, out_hbm,
                sidx, didx, hrows, erows, aggsh, gsem, esem, isem):
  c = lax.axis_index("c")
  s = lax.axis_index("s")
  wid = c * NS + s
  ebase = wid * EPW

  zcp = pltpu.async_copy(z_hbm, aggsh.at[pl.ds(s * RPT, RPT)], gsem[0])

  pltpu.sync_copy(src_hbm.at[pl.ds(ebase, CH)], sidx.at[0])
  pltpu.sync_copy(dst_hbm.at[pl.ds(ebase, CH)], didx.at[0])
  pltpu.async_copy(src_hbm.at[pl.ds(ebase + CH, CH)], sidx.at[1], isem[1])
  pltpu.async_copy(dst_hbm.at[pl.ds(ebase + CH, CH)], didx.at[1], isem[1])
  zcp.wait()
  pltpu.async_copy(h_hbm.at[sidx.at[0]], hrows.at[0], gsem[0])
  pltpu.async_copy(e_hbm.at[pl.ds(ebase, CH)], erows.at[0], esem[0])
  plsc.subcore_barrier()

  def _when(cond):
    def deco(fn):
      if isinstance(cond, (bool, int)):
        if cond:
          fn()
        return fn
      return pl.when(cond)(fn)
    return deco

  def _do_chunk(t, b):
    b2 = 1 - b

    @_when(t + 1 < NCHUNK)
    def _():
      pltpu.make_async_copy(src_hbm.at[pl.ds(0, CH)], sidx.at[b2],
                            isem[b2]).wait()
      pltpu.make_async_copy(dst_hbm.at[pl.ds(0, CH)], didx.at[b2],
                            isem[b2]).wait()
      pltpu.async_copy(h_hbm.at[sidx.at[b2]], hrows.at[b2], gsem[b2])
      pltpu.async_copy(e_hbm.at[pl.ds(ebase + (t + 1) * CH, CH)],
                       erows.at[b2], esem[b2])

    pltpu.make_async_copy(h_hbm.at[pl.ds(0, CH)], hrows.at[b], gsem[b]).wait()
    pltpu.make_async_copy(e_hbm.at[pl.ds(0, CH)], erows.at[b], esem[b]).wait()

    @pl.loop(0, CH)
    def _row(r):
      for j in range(EMB // 16):
        sl = pl.ds(j * 16, 16)
        hrows[b, r, sl] = jnp.maximum(hrows[b, r, sl] + erows[b, r, sl], 0.0)

    pltpu.sync_copy(hrows.at[b], aggsh.at[didx.at[b]], add=True)

    @_when(t + 2 < NCHUNK)
    def _():
      nbase = ebase + (t + 2) * CH
      pltpu.async_copy(src_hbm.at[pl.ds(nbase, CH)], sidx.at[b], isem[b])
      pltpu.async_copy(dst_hbm.at[pl.ds(nbase, CH)], didx.at[b], isem[b])

  @pl.loop(0, NCHUNK // 2)
  def _pair(i):
    t0 = 2 * i
    _do_chunk(t0, 0)
    _do_chunk(t0 + 1, 1)

  if NCHUNK % 2:
    _do_chunk(NCHUNK - 1, 0)

  plsc.subcore_barrier()
  pltpu.sync_copy(aggsh.at[pl.ds(s * RPT, RPT)],
                  out_hbm.at[pl.ds(c * NPAD + s * RPT, RPT)])


def _mlp(h, parts, alpha, W1f, b1f, W2f, b2f, relu_out):
  def body(al_ref, h_ref, p_ref, w1_ref, b1_ref, w2_ref, b2_ref, o_ref):
    t = h_ref[...] * al_ref[0, 0] + p_ref[0] + p_ref[1]
    t = jnp.dot(t, w1_ref[...], preferred_element_type=jnp.float32) + b1_ref[...]
    t = jnp.maximum(t, 0.0)
    t = jnp.dot(t, w2_ref[...], preferred_element_type=jnp.float32) + b2_ref[...]
    if relu_out:
      t = jnp.maximum(t, 0.0)
    o_ref[...] = t

  return pl.pallas_call(
      body,
      grid=(N_NODES // _RB,),
      in_specs=[
          pl.BlockSpec((1, 1), lambda i: (0, 0)),
          pl.BlockSpec((_RB, EMB), lambda i: (i, 0)),
          pl.BlockSpec((NC, _RB, EMB), lambda i: (0, i, 0)),
          pl.BlockSpec((EMB, 2 * EMB), lambda i: (0, 0)),
          pl.BlockSpec((1, 2 * EMB), lambda i: (0, 0)),
          pl.BlockSpec((2 * EMB, EMB), lambda i: (0, 0)),
          pl.BlockSpec((1, EMB), lambda i: (0, 0)),
      ],
      out_specs=pl.BlockSpec((_RB, EMB), lambda i: (i, 0)),
      out_shape=jax.ShapeDtypeStruct((N_NODES, EMB), jnp.float32),
  )(alpha, h, parts, W1f, b1f[None], W2f, b2f[None])


def kernel(x, edge_index, edge_attr, We, be, eps, W1, b1, W2, b2,
           g1, bb1, m1, v1, go, bo, mo, vo):
  s1 = g1 / jnp.sqrt(v1 + 1e-5)
  W1f = W1 * s1[:, None, :]
  b1f = (b1 - m1) * s1 + bb1
  so = go / jnp.sqrt(vo + 1e-5)
  W2f = W2 * so[:, None, :]
  b2f = (b2 - mo) * so + bo

  Wcat = jnp.concatenate([We[0], We[1]], axis=1)
  bcat = jnp.concatenate([be[0], be[1]])[None, :]
  E1, E2 = _edge_embed(edge_attr, Wcat, bcat)

  src = edge_index[0]
  dst = edge_index[1]
  z = jnp.zeros((RPT, EMB), jnp.float32)

  h = x
  for l in range(2):
    El = E1 if l == 0 else E2
    parts = _sc_segment(h, El, src, dst, z).reshape(NC, NPAD, EMB)
    alpha = (1.0 + eps[l]).reshape(1, 1)
    h = _mlp(h, parts, alpha, W1f[l], b1f[l], W2f[l], b2f[l],
             relu_out=(l == 0))
  return h
